# baseline (speedup 1.0000x reference)
import functools

import jax
import jax.numpy as jnp
from jax import lax
from jax.experimental import pallas as pl
from jax.experimental.pallas import tpu as pltpu
from jax.experimental.pallas import tpu_sc as plsc

DIM = 64
BATCH = 4
SEQ = 8192
NTOK = BATCH * SEQ
NW = 32
BPW = NTOK // NW
WPB = SEQ // BPW
NB = 4
CH = BPW // NB
NCH = BPW // CH

_mesh = plsc.VectorSubcoreMesh(core_axis_name="c", subcore_axis_name="s")


@functools.partial(
    pl.kernel,
    mesh=_mesh,
    out_type=jax.ShapeDtypeStruct((BATCH, SEQ, DIM), jnp.float32),
    compiler_params=pltpu.CompilerParams(use_tc_tiling_on_sc=False),
    scratch_types=(
        [pltpu.VMEM((BPW,), jnp.int32)]
        + [pltpu.VMEM((CH, DIM), jnp.float32) for _ in range(NB)]
        + [pltpu.SemaphoreType.DMA for _ in range(3 * NB)]
    ),
)
def _embed(ids_hbm, tok_hbm, pos_hbm, out_hbm, idx_v, *bufsems):
    rows = bufsems[:NB]
    sem_pos = bufsems[NB:2 * NB]
    sem_tok = bufsems[2 * NB:3 * NB]
    sem_out = bufsems[3 * NB:]
    wid = lax.axis_index("s") * 2 + lax.axis_index("c")
    b = lax.div(wid, WPB)
    col = lax.rem(wid, WPB) * BPW
    pltpu.sync_copy(ids_hbm.at[b, pl.ds(col, BPW)], idx_v)

    pos_dma = [None] * NCH
    tok_dma = [None] * NCH
    out_dma = [None] * NCH
    for c in range(NCH):
        pos_dma[c] = pltpu.async_copy(
            pos_hbm.at[pl.ds(col + c * CH, CH)], rows[c % NB],
            sem_pos[c % NB])
    for c in range(NCH):
        pos_dma[c].wait()
        tok_dma[c] = pltpu.async_copy(
            tok_hbm.at[idx_v.at[pl.ds(c * CH, CH)]], rows[c % NB],
            sem_tok[c % NB], add=True)
    for c in range(NCH):
        tok_dma[c].wait()
        out_dma[c] = pltpu.async_copy(
            rows[c % NB], out_hbm.at[b, pl.ds(col + c * CH, CH)],
            sem_out[c % NB])
    for c in range(NCH):
        out_dma[c].wait()


def kernel(input_ids, token_table, pos_table):
    return _embed(input_ids.astype(jnp.int32), token_table, pos_table)


# speedup vs baseline: 1.0477x; 1.0477x over previous
import functools

import jax
import jax.numpy as jnp
from jax import lax
from jax.experimental import pallas as pl
from jax.experimental.pallas import tpu as pltpu
from jax.experimental.pallas import tpu_sc as plsc

DIM = 64
PDIM = 128
BATCH = 4
SEQ = 8192
NTOK = BATCH * SEQ
NW = 32
BPW = NTOK // NW
NB = 4
CH = 128
NCH = BPW // CH

_mesh = plsc.VectorSubcoreMesh(core_axis_name="c", subcore_axis_name="s")


@functools.partial(
    pl.kernel,
    mesh=_mesh,
    out_type=jax.ShapeDtypeStruct((NTOK, PDIM), jnp.float32),
    scratch_types=(
        [pltpu.VMEM((CH,), jnp.int32) for _ in range(NB)]
        + [pltpu.VMEM((CH, PDIM), jnp.float32) for _ in range(NB)]
        + [pltpu.SemaphoreType.DMA for _ in range(3 * NB)]
    ),
)
def _embed(ids_hbm, tok_hbm, pos_hbm, out_hbm, *bufsems):
    idx = bufsems[:NB]
    rows = bufsems[NB:2 * NB]
    sem_pos = bufsems[2 * NB:3 * NB]
    sem_tok = bufsems[3 * NB:4 * NB]
    sem_out = bufsems[4 * NB:]
    wid = lax.axis_index("s") * 2 + lax.axis_index("c")
    base = wid * BPW
    pos_base = lax.rem(base, SEQ)

    pos_dma = [None] * NCH
    tok_dma = [None] * NCH
    out_dma = [None] * NCH
    for c in range(NB):
        pos_dma[c] = pltpu.async_copy(
            pos_hbm.at[pl.ds(pos_base + c * CH, CH)], rows[c % NB],
            sem_pos[c % NB])
        pltpu.sync_copy(ids_hbm.at[pl.ds(base + c * CH, CH)], idx[c % NB])
    for c in range(NCH):
        if c >= NB:
            out_dma[c - NB].wait()
            pos_dma[c] = pltpu.async_copy(
                pos_hbm.at[pl.ds(pos_base + c * CH, CH)], rows[c % NB],
                sem_pos[c % NB])
            pltpu.sync_copy(ids_hbm.at[pl.ds(base + c * CH, CH)], idx[c % NB])
        pos_dma[c].wait()
        tok_dma[c] = pltpu.async_copy(
            tok_hbm.at[idx[c % NB]], rows[c % NB],
            sem_tok[c % NB], add=True)
        tok_dma[c].wait()
        out_dma[c] = pltpu.async_copy(
            rows[c % NB], out_hbm.at[pl.ds(base + c * CH, CH)],
            sem_out[c % NB])
    for c in range(NCH - NB, NCH):
        out_dma[c].wait()


def kernel(input_ids, token_table, pos_table):
    ids = input_ids.reshape(NTOK).astype(jnp.int32)
    tok128 = jnp.pad(token_table, ((0, 0), (0, PDIM - DIM)))
    pos128 = jnp.pad(pos_table, ((0, 0), (0, PDIM - DIM)))
    out = _embed(ids, tok128, pos128)
    return out[:, :DIM].reshape(BATCH, SEQ, DIM)
